# Initial kernel scaffold; baseline (speedup 1.0000x reference)
import jax
import jax.numpy as jnp
from jax.experimental import pallas as pl


def kernel(feature_matrix, edge_index, w):
    raise NotImplementedError("write your pallas kernel here")


# speedup vs baseline: 5.5410x; 5.5410x over previous
import functools

import jax
import jax.numpy as jnp
from jax import lax
from jax.experimental import pallas as pl
from jax.experimental.pallas import tpu as pltpu
from jax.experimental.pallas import tpu_sc as plsc

N = 10000
E = 320000
D = 128

NC = 2
NS = 16
NW = NC * NS
E_PER_TILE = E // NW
CHUNK = 80
N_CHUNKS = E_PER_TILE // CHUNK
ROWS_A = 624
ROWS_LAST = N - (NS - 1) * ROWS_A


def _sc_body(x_hbm, src_hbm, dst_hbm, zeros_hbm, part_hbm,
             sidx, didx, rows, acc, gsem):
    c = lax.axis_index("c")
    s = lax.axis_index("s")
    wid = c * NS + s

    @pl.when(s < NS - 1)
    def _():
        pltpu.sync_copy(zeros_hbm.at[pl.ds(0, ROWS_A)],
                        acc.at[pl.ds(s * ROWS_A, ROWS_A)])

    @pl.when(s == NS - 1)
    def _():
        pltpu.sync_copy(zeros_hbm,
                        acc.at[pl.ds((NS - 1) * ROWS_A, ROWS_LAST)])

    plsc.subcore_barrier()

    base = wid * E_PER_TILE

    @pl.loop(0, N_CHUNKS)
    def _chunk(i):
        off = base + i * CHUNK
        pltpu.sync_copy(src_hbm.at[pl.ds(off, CHUNK)], sidx)
        pltpu.sync_copy(dst_hbm.at[pl.ds(off, CHUNK)], didx)
        pltpu.async_copy(x_hbm.at[sidx], rows, gsem).wait()
        pltpu.sync_copy(rows, acc.at[didx], add=True)

    plsc.subcore_barrier()

    @pl.when(s < NS - 1)
    def _():
        r0 = s * ROWS_A
        pltpu.sync_copy(acc.at[pl.ds(r0, ROWS_A)],
                        part_hbm.at[pl.ds(c * N + r0, ROWS_A)])

    @pl.when(s == NS - 1)
    def _():
        r0 = (NS - 1) * ROWS_A
        pltpu.sync_copy(acc.at[pl.ds(r0, ROWS_LAST)],
                        part_hbm.at[pl.ds(c * N + r0, ROWS_LAST)])


_sc_segment_sum = functools.partial(
    pl.kernel,
    out_type=jax.ShapeDtypeStruct((NC * N, D), jnp.float32),
    mesh=plsc.VectorSubcoreMesh(core_axis_name="c", subcore_axis_name="s"),
    scratch_types=[
        pltpu.VMEM((CHUNK,), jnp.int32),
        pltpu.VMEM((CHUNK,), jnp.int32),
        pltpu.VMEM((CHUNK, D), jnp.float32),
        pltpu.VMEM_SHARED((N, D), jnp.float32),
        pltpu.SemaphoreType.DMA,
    ],
)(_sc_body)


def _combine_body(p0_ref, p1_ref, w_ref, o_ref):
    o_ref[...] = (p0_ref[...] + p1_ref[...]) * w_ref[...]


_BLK = 2000


def _combine(partials, w):
    return pl.pallas_call(
        _combine_body,
        grid=(N // _BLK,),
        in_specs=[
            pl.BlockSpec((_BLK, D), lambda i: (i, 0)),
            pl.BlockSpec((_BLK, D), lambda i: (i + N // _BLK, 0)),
            pl.BlockSpec((1, D), lambda i: (0, 0)),
        ],
        out_specs=pl.BlockSpec((_BLK, D), lambda i: (i, 0)),
        out_shape=jax.ShapeDtypeStruct((N, D), jnp.float32),
    )(partials, partials, w)


@jax.jit
def kernel(feature_matrix, edge_index, w):
    x = jnp.squeeze(feature_matrix)
    src = edge_index[0]
    dst = edge_index[1]
    zeros = jnp.zeros((ROWS_LAST, D), jnp.float32)
    partials = _sc_segment_sum(x, src, dst, zeros)
    return _combine(partials, w)


# speedup vs baseline: 14.5184x; 2.6202x over previous
import functools

import jax
import jax.numpy as jnp
from jax import lax
from jax.experimental import pallas as pl
from jax.experimental.pallas import tpu as pltpu
from jax.experimental.pallas import tpu_sc as plsc

N = 10000
E = 320000
D = 128

NC = 2
NS = 16
NW = NC * NS

CHUNK = 80
E_PER_TILE = E // NW
N_CHUNKS = E_PER_TILE // CHUNK
NBUF = 3

ROWS_A = 624
ROWS_LAST = N - (NS - 1) * ROWS_A


def _sc_body(x_hbm, src_hbm, dst_hbm, part_hbm,
             sidx, didx, rows, acc, gsems):
    c = lax.axis_index("c")
    s = lax.axis_index("s")
    wid = c * NS + s

    pltpu.sync_copy(src_hbm.at[wid], sidx)
    pltpu.sync_copy(dst_hbm.at[wid], didx)

    zvec = jnp.zeros((16,), jnp.float32)

    @pl.loop(0, CHUNK)
    def _zrow(r):
        for j in range(D // 16):
            rows[0][r, pl.ds(j * 16, 16)] = zvec

    for q in range(ROWS_A // CHUNK):
        pltpu.sync_copy(rows[0], acc.at[pl.ds(s * ROWS_A + q * CHUNK, CHUNK)])
    _rem = ROWS_A % CHUNK

    @pl.when(s < NS - 1)
    def _():
        pltpu.sync_copy(
            rows[0].at[pl.ds(0, _rem)],
            acc.at[pl.ds(s * ROWS_A + ROWS_A - _rem, _rem)])

    @pl.when(s == NS - 1)
    def _():
        pltpu.sync_copy(rows[0],
                        acc.at[pl.ds(N - CHUNK, CHUNK)])

    plsc.subcore_barrier()

    def gather(j, b):
        pltpu.async_copy(x_hbm.at[sidx.at[pl.ds(j * CHUNK, CHUNK)]],
                         rows[b], gsems[b])

    def gather_wait(j, b):
        pltpu.make_async_copy(x_hbm.at[sidx.at[pl.ds(j * CHUNK, CHUNK)]],
                              rows[b], gsems[b]).wait()

    def scatter(j, b):
        pltpu.sync_copy(rows[b], acc.at[didx.at[pl.ds(j * CHUNK, CHUNK)]],
                        add=True)

    def step(j, b):
        gather_wait(j, b)
        scatter(j, b)

        @pl.when(j + NBUF < N_CHUNKS)
        def _():
            gather(j + NBUF, b)

    for j in range(NBUF):
        gather(j, j)

    @pl.loop(0, N_CHUNKS - 2, step=NBUF)
    def _group(i):
        for k in range(NBUF):
            step(i + k, k)

    step(N_CHUNKS - 2, (N_CHUNKS - 2) % NBUF)
    step(N_CHUNKS - 1, (N_CHUNKS - 1) % NBUF)

    plsc.subcore_barrier()

    @pl.when(s < NS - 1)
    def _():
        r0 = s * ROWS_A
        pltpu.sync_copy(acc.at[pl.ds(r0, ROWS_A)],
                        part_hbm.at[pl.ds(c * N + r0, ROWS_A)])

    @pl.when(s == NS - 1)
    def _():
        r0 = (NS - 1) * ROWS_A
        pltpu.sync_copy(acc.at[pl.ds(r0, ROWS_LAST)],
                        part_hbm.at[pl.ds(c * N + r0, ROWS_LAST)])


_sc_segment_sum = functools.partial(
    pl.kernel,
    out_type=jax.ShapeDtypeStruct((NC * N, D), jnp.float32),
    mesh=plsc.VectorSubcoreMesh(core_axis_name="c", subcore_axis_name="s"),
    scratch_types=[
        pltpu.VMEM((E_PER_TILE,), jnp.int32),
        pltpu.VMEM((E_PER_TILE,), jnp.int32),
        [pltpu.VMEM((CHUNK, D), jnp.float32) for _ in range(NBUF)],
        pltpu.VMEM_SHARED((N, D), jnp.float32),
        [pltpu.SemaphoreType.DMA for _ in range(NBUF)],
    ],
)(_sc_body)


def _combine_body(p0_ref, p1_ref, w_ref, o_ref):
    o_ref[...] = (p0_ref[...] + p1_ref[...]) * w_ref[...]


_BLK = 2000


def _combine(partials, w):
    return pl.pallas_call(
        _combine_body,
        grid=(N // _BLK,),
        in_specs=[
            pl.BlockSpec((_BLK, D), lambda i: (i, 0)),
            pl.BlockSpec((_BLK, D), lambda i: (i + N // _BLK, 0)),
            pl.BlockSpec((1, D), lambda i: (0, 0)),
        ],
        out_specs=pl.BlockSpec((_BLK, D), lambda i: (i, 0)),
        out_shape=jax.ShapeDtypeStruct((N, D), jnp.float32),
    )(partials, partials, w)


@jax.jit
def kernel(feature_matrix, edge_index, w):
    x = jnp.squeeze(feature_matrix)
    src = edge_index[0].reshape(NW, E_PER_TILE)
    dst = edge_index[1].reshape(NW, E_PER_TILE)
    partials = _sc_segment_sum(x, src, dst)
    return _combine(partials, w)
